# Initial kernel scaffold; baseline (speedup 1.0000x reference)
import jax
import jax.numpy as jnp
from jax.experimental import pallas as pl


def kernel(x, edge_index, id_index, query_kernel, query_bias, key_kernel, key_bias, kernel, kernel_id, bias):
    raise NotImplementedError("write your pallas kernel here")


# speedup vs baseline: 21.9197x; 21.9197x over previous
import functools

import jax
import jax.numpy as jnp
from jax import lax
from jax.experimental import pallas as pl
from jax.experimental.pallas import tpu as pltpu
from jax.experimental.pallas import tpu_sc as plsc

NC, NS, L = 2, 16, 16
NW = NC * NS
F = 128
NH = 8
DH = F // NH
C = 64


def _tc_qkv(xp, xs, wqkv, wid, bq, bk, n1):
    bm = 1024

    def body(x_ref, xs_ref, w_ref, wid_ref, bq_ref, bk_ref, q_ref, kv_ref):
        u = jnp.dot(x_ref[...], w_ref[...], preferred_element_type=jnp.float32)
        q_ref[...] = jnp.maximum(u[:, :F] + bq_ref[...], 0.0)
        kk = jnp.maximum(u[:, F:2 * F] + bk_ref[...], 0.0)
        vv = u[:, 2 * F:] + jnp.dot(
            xs_ref[...], wid_ref[...], preferred_element_type=jnp.float32)
        kv_ref[...] = jnp.concatenate([kk, vv], axis=1)

    return pl.pallas_call(
        body,
        grid=(n1 // bm,),
        in_specs=[
            pl.BlockSpec((bm, F), lambda i: (i, 0)),
            pl.BlockSpec((bm, F), lambda i: (i, 0)),
            pl.BlockSpec((F, 3 * F), lambda i: (0, 0)),
            pl.BlockSpec((F, F), lambda i: (0, 0)),
            pl.BlockSpec((1, F), lambda i: (0, 0)),
            pl.BlockSpec((1, F), lambda i: (0, 0)),
        ],
        out_specs=[pl.BlockSpec((bm, F), lambda i: (i, 0)),
                   pl.BlockSpec((bm, 2 * F), lambda i: (i, 0))],
        out_shape=[jax.ShapeDtypeStruct((n1, F), jnp.float32),
                   jax.ShapeDtypeStruct((n1, 2 * F), jnp.float32)],
    )(xp, xs, wqkv, wid, bq, bk)


def _sc_xs(xp, idp, z128, n1, idp_len):
    rpt = n1 // NS
    cpt = idp_len // NS
    nch = cpt // C
    mesh = plsc.VectorSubcoreMesh(core_axis_name="c", subcore_axis_name="s")

    @functools.partial(
        pl.kernel,
        out_type=jax.ShapeDtypeStruct((n1, F), jnp.float32),
        mesh=mesh,
        scratch_types=[
            pltpu.VMEM_SHARED((n1, F), jnp.float32),
            pltpu.VMEM((C, F), jnp.float32),
            pltpu.VMEM((C,), jnp.int32),
            pltpu.SemaphoreType.DMA,
        ],
    )
    def run(x_hbm, id_hbm, z_hbm, xs_hbm, acc_sh, rows_v, idx_v, sem):
        cid = lax.axis_index("c")
        sid = lax.axis_index("s")

        @pl.when(cid == 0)
        def _():
            pltpu.sync_copy(z_hbm, acc_sh.at[pl.ds(sid * rpt, rpt)])
            plsc.subcore_barrier()

            def chunk(ci, carry):
                pltpu.sync_copy(id_hbm.at[pl.ds(sid * cpt + ci * C, C)], idx_v)
                pltpu.async_copy(x_hbm.at[idx_v], rows_v, sem).wait()
                pltpu.sync_copy(rows_v, acc_sh.at[idx_v], add=True)
                return carry

            lax.fori_loop(0, nch, chunk, 0)
            plsc.subcore_barrier()
            pltpu.sync_copy(acc_sh.at[pl.ds(sid * rpt, rpt)],
                            xs_hbm.at[pl.ds(sid * rpt, rpt)])

    return run(xp, idp, z128)


def _sc_edge(q, kv, ridx, cidx, z128, z16, n1, ep):
    rpt = n1 // NS
    ept = ep // NW
    nchunk = ept // C
    ng = C // L
    mesh = plsc.VectorSubcoreMesh(core_axis_name="c", subcore_axis_name="s")

    @functools.partial(
        pl.kernel,
        out_type=(jax.ShapeDtypeStruct((NC, n1, F), jnp.float32),
                  jax.ShapeDtypeStruct((NC, n1, L), jnp.float32)),
        mesh=mesh,
        scratch_types=[
            pltpu.VMEM_SHARED((n1, F), jnp.float32),
            pltpu.VMEM_SHARED((n1, L), jnp.float32),
            pltpu.VMEM((C, F), jnp.float32),
            pltpu.VMEM((C, 2 * F), jnp.float32),
            pltpu.VMEM((C, F), jnp.float32),
            pltpu.VMEM((C, L), jnp.float32),
            pltpu.VMEM((C,), jnp.int32),
            pltpu.VMEM((C,), jnp.int32),
            pltpu.SemaphoreType.DMA,
            pltpu.SemaphoreType.DMA,
        ],
        compiler_params=pltpu.CompilerParams(needs_layout_passes=False,
                                             use_tc_tiling_on_sc=False),
    )
    def run(q_hbm, kv_hbm, r_hbm, c_hbm, z128_hbm, z16_hbm,
            a128_hbm, a16_hbm,
            acc_sh, accd_sh, qb, kvb, vb, pb, rix, cix, sem0, sem1):
        cid = lax.axis_index("c")
        sid = lax.axis_index("s")
        wid = sid * NC + cid

        pltpu.sync_copy(z128_hbm, acc_sh.at[pl.ds(sid * rpt, rpt)])
        pltpu.sync_copy(z16_hbm, accd_sh.at[pl.ds(sid * rpt, rpt)])
        zv = jnp.zeros((L,), jnp.float32)

        def zrow(e, carry):
            pb[e, :] = zv
            return carry

        lax.fori_loop(0, C, zrow, 0)
        plsc.subcore_barrier()

        e_base = jnp.arange(L, dtype=jnp.int32)
        base0 = wid * ept

        def chunk(ci, carry):
            b = base0 + ci * C
            pltpu.sync_copy(r_hbm.at[pl.ds(b, C)], rix)
            pltpu.sync_copy(c_hbm.at[pl.ds(b, C)], cix)
            cp0 = pltpu.async_copy(q_hbm.at[rix], qb, sem0)
            cp1 = pltpu.async_copy(kv_hbm.at[cix], kvb, sem1)
            cp0.wait()
            cp1.wait()

            def group(g, gcarry):
                e_ids = g * L + e_base
                sgs = []
                for h in range(NH):
                    qfs = [plsc.load_gather(
                        qb, [e_ids, jnp.full((L,), h * DH + f, jnp.int32)])
                        for f in range(DH)]
                    kfs = [plsc.load_gather(
                        kvb, [e_ids, jnp.full((L,), h * DH + f, jnp.int32)])
                        for f in range(DH)]
                    prods = [qfs[f] * kfs[f] for f in range(DH)]
                    while len(prods) > 1:
                        prods = [prods[i] + prods[i + 1]
                                 for i in range(0, len(prods), 2)]
                    sg = jnp.exp(prods[0] * 0.25)
                    plsc.store_scatter(
                        pb, [e_ids, jnp.full((L,), h, jnp.int32)], sg)
                    sgs.append(sg)
                for h in range(NH):
                    vfs = [plsc.load_gather(
                        kvb, [e_ids, jnp.full((L,), F + h * DH + f, jnp.int32)])
                        for f in range(DH)]
                    mfs = [vfs[f] * sgs[h] for f in range(DH)]
                    for f in range(DH):
                        plsc.store_scatter(
                            vb, [e_ids, jnp.full((L,), h * DH + f, jnp.int32)],
                            mfs[f])
                return gcarry

            lax.fori_loop(0, ng, group, 0)
            pltpu.sync_copy(vb, acc_sh.at[rix], add=True)
            pltpu.sync_copy(pb, accd_sh.at[rix], add=True)
            return carry

        lax.fori_loop(0, nchunk, chunk, 0)
        plsc.subcore_barrier()
        pltpu.sync_copy(acc_sh.at[pl.ds(sid * rpt, rpt)],
                        a128_hbm.at[cid].at[pl.ds(sid * rpt, rpt)])
        pltpu.sync_copy(accd_sh.at[pl.ds(sid * rpt, rpt)],
                        a16_hbm.at[cid].at[pl.ds(sid * rpt, rpt)])

    return run(q, kv, ridx, cidx, z128, z16)


def _tc_out(a128, a16, b2, n1):
    bm = 1024

    def body(a_ref, d_ref, b_ref, o_ref):
        a = a_ref[0] + a_ref[1]
        d = d_ref[0] + d_ref[1]
        d8 = d[:, :NH]
        rep = (lax.broadcasted_iota(jnp.int32, (NH, F), 1) // DH ==
               lax.broadcasted_iota(jnp.int32, (NH, F), 0)).astype(jnp.float32)
        den = jnp.dot(d8, rep, preferred_element_type=jnp.float32)
        o_ref[...] = a / den + b_ref[...]

    return pl.pallas_call(
        body,
        grid=(n1 // bm,),
        in_specs=[
            pl.BlockSpec((NC, bm, F), lambda i: (0, i, 0)),
            pl.BlockSpec((NC, bm, L), lambda i: (0, i, 0)),
            pl.BlockSpec((1, F), lambda i: (0, 0)),
        ],
        out_specs=pl.BlockSpec((bm, F), lambda i: (i, 0)),
        out_shape=jax.ShapeDtypeStruct((n1, F), jnp.float32),
    )(a128, a16, b2)


def kernel(x, edge_index, id_index, query_kernel, query_bias, key_kernel,
           key_bias, kernel, kernel_id, bias):
    n, f = x.shape
    assert f == F
    e = edge_index.shape[1]
    nid = id_index.shape[0]

    n1 = ((n + 1 + 511) // 512) * 512
    et = e + n
    ep = ((et + NW * C - 1) // (NW * C)) * (NW * C)
    idp_len = ((nid + NS * C - 1) // (NS * C)) * (NS * C)

    xp = jnp.zeros((n1, F), jnp.float32).at[:n].set(x.astype(jnp.float32))
    loops = jnp.arange(n, dtype=jnp.int32)
    pad_e = jnp.full((ep - et,), n, jnp.int32)
    ridx = jnp.concatenate([edge_index[0].astype(jnp.int32), loops, pad_e])
    cidx = jnp.concatenate([edge_index[1].astype(jnp.int32), loops, pad_e])
    idp = jnp.concatenate(
        [id_index.astype(jnp.int32), jnp.full((idp_len - nid,), n, jnp.int32)])

    z128 = jnp.zeros((n1 // NS, F), jnp.float32)
    z16 = jnp.zeros((n1 // NS, L), jnp.float32)
    wqkv = jnp.concatenate([query_kernel, key_kernel, kernel], axis=1)

    xs = _sc_xs(xp, idp, z128, n1, idp_len)
    qt, kvt = _tc_qkv(xp, xs, wqkv, kernel_id,
                      query_bias[None, :], key_bias[None, :], n1)
    a128, a16 = _sc_edge(qt, kvt, ridx, cidx, z128, z16, n1, ep)
    out = _tc_out(a128, a16, bias[None, :], n1)
    return out[:n]
